# Initial kernel scaffold; baseline (speedup 1.0000x reference)
import jax
import jax.numpy as jnp
from jax.experimental import pallas as pl


def kernel(xb, yb, table):
    raise NotImplementedError("write your pallas kernel here")


# speedup vs baseline: 1.6097x; 1.6097x over previous
import functools

import jax
import jax.numpy as jnp
from jax import lax
from jax.experimental import pallas as pl
from jax.experimental.pallas import tpu as pltpu
from jax.experimental.pallas import tpu_sc as plsc

VOCAB = 1000
N = 204800
NC, NS, L = 2, 16, 16
NW = NC * NS
BPW = N // NW
CHUNK = 64
STEPS = BPW // CHUNK
GROUPS = BPW // L

_NEG = -1e30


def _lse_body(tab_ref, out_ref):
    x = tab_ref[...]
    m = jnp.max(x, axis=1, keepdims=True)
    s = jnp.sum(jnp.exp(x - m), axis=1, keepdims=True)
    lse = m + jnp.log(s)
    out_ref[...] = jax.lax.broadcast_in_dim(lse, (VOCAB, 128), (0, 1))


def _compute_lse(table_padded):
    return pl.pallas_call(
        _lse_body,
        out_shape=jax.ShapeDtypeStruct((VOCAB, 128), jnp.float32),
    )(table_padded)


_MESH = plsc.VectorSubcoreMesh(core_axis_name="c", subcore_axis_name="s")


@functools.partial(
    pl.kernel,
    mesh=_MESH,
    compiler_params=pltpu.CompilerParams(
        use_tc_tiling_on_sc=False, needs_layout_passes=False
    ),
    out_type=[
        jax.ShapeDtypeStruct((N, VOCAB), jnp.float32),
        jax.ShapeDtypeStruct((NW, L), jnp.float32),
    ],
    scratch_types=[
        pltpu.VMEM((BPW,), jnp.int32),
        pltpu.VMEM((BPW,), jnp.int32),
        pltpu.VMEM((BPW,), jnp.float32),
        pltpu.VMEM((CHUNK, VOCAB), jnp.float32),
        pltpu.VMEM((L,), jnp.float32),
        pltpu.SemaphoreType.DMA,
        pltpu.SemaphoreType.DMA,
    ],
)
def _sc_gather_loss(xb_hbm, yb_hbm, lse_hbm, table_hbm,
                    out_hbm, part_hbm,
                    xb_v, yb_v, lsex_v, rows_v, acc_v,
                    sem_r, sem_l):
    wid = lax.axis_index("s") * NC + lax.axis_index("c")
    base = wid * BPW
    pltpu.sync_copy(xb_hbm.at[pl.ds(base, BPW)], xb_v)
    pltpu.sync_copy(yb_hbm.at[pl.ds(base, BPW)], yb_v)

    lsex_dma = pltpu.async_copy(lse_hbm.at[xb_v], lsex_v, sem_l)

    def step(i, acc):
        off = base + i * CHUNK
        pltpu.async_copy(
            table_hbm.at[xb_v.at[pl.ds(i * CHUNK, CHUNK)]], rows_v, sem_r
        ).wait()
        pltpu.sync_copy(rows_v, out_hbm.at[pl.ds(off, CHUNK)])
        for j in range(CHUNK // L):
            ybv = yb_v[pl.ds(i * CHUNK + j * L, L)]
            rid = lax.iota(jnp.int32, L) + (j * L)
            acc = acc - plsc.load_gather(rows_v, [rid, ybv])
        return acc

    acc = lax.fori_loop(0, STEPS, step, jnp.zeros((L,), jnp.float32))

    lsex_dma.wait()

    def agroup(g, acc):
        return acc + lsex_v[pl.ds(g * L, L)]

    acc = lax.fori_loop(0, GROUPS, agroup, acc)
    acc_v[...] = acc
    pltpu.sync_copy(acc_v, part_hbm.at[wid])


def kernel(xb, yb, table):
    xb_flat = xb.reshape(N).astype(jnp.int32)
    yb_flat = yb.reshape(N).astype(jnp.int32)
    pad = jnp.full((VOCAB, 24), _NEG, dtype=jnp.float32)
    lse2d = _compute_lse(jnp.concatenate([table, pad], axis=1))
    lse = lse2d[:, 0]
    logits, parts = _sc_gather_loss(xb_flat, yb_flat, lse, table)
    loss = jnp.sum(parts) / jnp.float32(N)
    return (logits, loss)


# speedup vs baseline: 1.6326x; 1.0143x over previous
import functools

import jax
import jax.numpy as jnp
from jax import lax
from jax.experimental import pallas as pl
from jax.experimental.pallas import tpu as pltpu
from jax.experimental.pallas import tpu_sc as plsc

VOCAB = 1000
N = 204800
NC, NS, L = 2, 16, 16
NW = NC * NS
BPW = N // NW
CHUNK = 32
STEPS = BPW // CHUNK
NBUF = 2
GROUPS = BPW // L

_NEG = -1e30


def _lse_body(tab_ref, out_ref):
    x = tab_ref[...]
    m = jnp.max(x, axis=1, keepdims=True)
    s = jnp.sum(jnp.exp(x - m), axis=1, keepdims=True)
    lse = m + jnp.log(s)
    out_ref[...] = jax.lax.broadcast_in_dim(lse, (VOCAB, 128), (0, 1))


def _compute_lse(table_padded):
    return pl.pallas_call(
        _lse_body,
        out_shape=jax.ShapeDtypeStruct((VOCAB, 128), jnp.float32),
    )(table_padded)


_MESH = plsc.VectorSubcoreMesh(core_axis_name="c", subcore_axis_name="s")


@functools.partial(
    pl.kernel,
    mesh=_MESH,
    compiler_params=pltpu.CompilerParams(
        use_tc_tiling_on_sc=False, needs_layout_passes=False
    ),
    out_type=[
        jax.ShapeDtypeStruct((N, VOCAB), jnp.float32),
        jax.ShapeDtypeStruct((NW, L), jnp.float32),
    ],
    scratch_types=[
        pltpu.VMEM((BPW,), jnp.int32),
        pltpu.VMEM((BPW,), jnp.int32),
        pltpu.VMEM((BPW,), jnp.float32),
        pltpu.VMEM((CHUNK, VOCAB), jnp.float32),
        pltpu.VMEM((CHUNK, VOCAB), jnp.float32),
        pltpu.VMEM((L,), jnp.float32),
        pltpu.SemaphoreType.DMA,
        pltpu.SemaphoreType.DMA,
        pltpu.SemaphoreType.DMA,
    ],
)
def _sc_gather_loss(xb_hbm, yb_hbm, lse_hbm, table_hbm,
                    out_hbm, part_hbm,
                    xb_v, yb_v, lsex_v, rows0_v, rows1_v, acc_v,
                    sem_r0, sem_r1, sem_l):
    wid = lax.axis_index("s") * NC + lax.axis_index("c")
    base = wid * BPW
    pltpu.sync_copy(xb_hbm.at[pl.ds(base, BPW)], xb_v)
    pltpu.sync_copy(yb_hbm.at[pl.ds(base, BPW)], yb_v)

    lsex_dma = pltpu.async_copy(lse_hbm.at[xb_v], lsex_v, sem_l)

    bufs = (rows0_v, rows1_v)
    sems = (sem_r0, sem_r1)

    def gather_rows(i, buf, sem):
        return pltpu.async_copy(
            table_hbm.at[xb_v.at[pl.ds(i * CHUNK, CHUNK)]], buf, sem
        )

    for b in range(NBUF):
        gather_rows(b, bufs[b], sems[b])

    def pair(g, acc):
        for b in range(NBUF):
            i = g * NBUF + b
            buf, sem = bufs[b], sems[b]
            pltpu.make_async_copy(
                table_hbm.at[xb_v.at[pl.ds(i * CHUNK, CHUNK)]], buf, sem
            ).wait()
            pltpu.sync_copy(buf, out_hbm.at[pl.ds(base + i * CHUNK, CHUNK)])
            for j in range(CHUNK // L):
                ybv = yb_v[pl.ds(i * CHUNK + j * L, L)]
                rid = lax.iota(jnp.int32, L) + (j * L)
                acc = acc - plsc.load_gather(buf, [rid, ybv])

            @pl.when(i + NBUF < STEPS)
            def _():
                gather_rows(i + NBUF, buf, sem)
        return acc

    acc = lax.fori_loop(0, STEPS // NBUF, pair, jnp.zeros((L,), jnp.float32))

    lsex_dma.wait()

    def agroup(g, acc):
        return acc + lsex_v[pl.ds(g * L, L)]

    acc = lax.fori_loop(0, GROUPS, agroup, acc)
    acc_v[...] = acc
    pltpu.sync_copy(acc_v, part_hbm.at[wid])


def kernel(xb, yb, table):
    xb_flat = xb.reshape(N).astype(jnp.int32)
    yb_flat = yb.reshape(N).astype(jnp.int32)
    pad = jnp.full((VOCAB, 24), _NEG, dtype=jnp.float32)
    lse2d = _compute_lse(jnp.concatenate([table, pad], axis=1))
    lse = lse2d[:, 0]
    logits, parts = _sc_gather_loss(xb_flat, yb_flat, lse, table)
    loss = jnp.sum(parts) / jnp.float32(N)
    return (logits, loss)
